# Initial kernel scaffold; baseline (speedup 1.0000x reference)
import jax
import jax.numpy as jnp
from jax.experimental import pallas as pl


def kernel(energy, batch):
    raise NotImplementedError("write your pallas kernel here")


# speedup vs baseline: 22.1780x; 22.1780x over previous
import functools

import jax
import jax.numpy as jnp
from jax import lax
from jax.experimental import pallas as pl
from jax.experimental.pallas import tpu as pltpu
from jax.experimental.pallas import tpu_sc as plsc

N_ATOMS = 1600000
N_SEG = 50000

NC = 2
NS = 16
NW = NC * NS

ROW = 128
ROWS_PER_TILE = 392
N_PAD = NW * ROWS_PER_TILE * ROW
ACC_PER_TILE = 3200
ACC = NS * ACC_PER_TILE


def _sc_body(e_hbm, b_hbm, out_hbm, accum, zbuf, idx2d, val2d, sem):
    c = lax.axis_index("c")
    s = lax.axis_index("s")
    wid = s * NC + c
    base_row = wid * ROWS_PER_TILE

    cp_b = pltpu.async_copy(b_hbm.at[pl.ds(base_row, ROWS_PER_TILE)], idx2d, sem)
    cp_e = pltpu.async_copy(e_hbm.at[pl.ds(base_row, ROWS_PER_TILE)], val2d, sem)

    def _zero(i, carry):
        zbuf[pl.ds(i * 16, 16)] = jnp.zeros((16,), jnp.float32)
        return carry

    lax.fori_loop(0, ACC_PER_TILE // 16, _zero, 0)
    pltpu.sync_copy(zbuf, accum.at[pl.ds(s * ACC_PER_TILE, ACC_PER_TILE)])
    plsc.subcore_barrier()

    cp_b.wait()
    cp_e.wait()

    def _scat(j, carry):
        pltpu.sync_copy(val2d.at[j], accum.at[idx2d.at[j]], add=True)
        return carry

    lax.fori_loop(0, ROWS_PER_TILE, _scat, 0)
    plsc.subcore_barrier()

    pltpu.sync_copy(
        accum.at[pl.ds(s * ACC_PER_TILE, ACC_PER_TILE)],
        out_hbm.at[c, pl.ds(s * ACC_PER_TILE, ACC_PER_TILE)],
    )


_sc_kernel = functools.partial(
    pl.kernel,
    out_type=jax.ShapeDtypeStruct((NC, ACC), jnp.float32),
    mesh=plsc.VectorSubcoreMesh(
        core_axis_name="c", subcore_axis_name="s", num_cores=NC, num_subcores=NS
    ),
    scratch_types=[
        pltpu.VMEM_SHARED((ACC,), jnp.float32),
        pltpu.VMEM((ACC_PER_TILE,), jnp.float32),
        pltpu.VMEM((ROWS_PER_TILE, ROW), jnp.int32),
        pltpu.VMEM((ROWS_PER_TILE, ROW), jnp.float32),
        pltpu.SemaphoreType.DMA,
    ],
)(_sc_body)


def _tc_add(x_ref, o_ref):
    o_ref[...] = x_ref[0] + x_ref[1]


@jax.jit
def kernel(energy, batch):
    pad = N_PAD - N_ATOMS
    e = jnp.concatenate([energy, jnp.zeros((pad,), jnp.float32)])
    b = jnp.concatenate([batch, jnp.arange(pad, dtype=jnp.int32) % N_SEG])
    partials = _sc_kernel(
        e.reshape(NW * ROWS_PER_TILE, ROW), b.reshape(NW * ROWS_PER_TILE, ROW)
    )
    out = pl.pallas_call(
        _tc_add,
        out_shape=jax.ShapeDtypeStruct((ACC // ROW, ROW), jnp.float32),
    )(partials.reshape(NC, ACC // ROW, ROW))
    return out.reshape(ACC)[:N_SEG]


# speedup vs baseline: 25.0733x; 1.1305x over previous
import functools

import jax
import jax.numpy as jnp
from jax import lax
from jax.experimental import pallas as pl
from jax.experimental.pallas import tpu as pltpu
from jax.experimental.pallas import tpu_sc as plsc

N_ATOMS = 1600000
N_SEG = 50000

NC = 2
NS = 16
NW = NC * NS

ROW = 128
ROWS_PER_TILE = 392
N_PAD = NW * ROWS_PER_TILE * ROW
ACC_PER_TILE = 3200
ACC = NS * ACC_PER_TILE


def _sc_body(e_hbm, b_hbm, out_hbm, accum, zbuf, idx2d, val2d, sem):
    c = lax.axis_index("c")
    s = lax.axis_index("s")
    wid = s * NC + c
    base_row = wid * ROWS_PER_TILE

    cp_b = pltpu.async_copy(b_hbm.at[pl.ds(base_row, ROWS_PER_TILE)], idx2d, sem)
    cp_e = pltpu.async_copy(e_hbm.at[pl.ds(base_row, ROWS_PER_TILE)], val2d, sem)

    def _zero(i, carry):
        zbuf[pl.ds(i * 16, 16)] = jnp.zeros((16,), jnp.float32)
        return carry

    lax.fori_loop(0, ACC_PER_TILE // 16, _zero, 0)
    pltpu.sync_copy(zbuf, accum.at[pl.ds(s * ACC_PER_TILE, ACC_PER_TILE)])
    plsc.subcore_barrier()

    cp_b.wait()
    cp_e.wait()

    K = 8

    def _scat(j, carry):
        base = j * K
        cps = [
            pltpu.async_copy(
                val2d.at[base + i], accum.at[idx2d.at[base + i]], sem, add=True
            )
            for i in range(K)
        ]
        for cp in cps:
            cp.wait()
        return carry

    lax.fori_loop(0, ROWS_PER_TILE // K, _scat, 0)
    plsc.subcore_barrier()

    pltpu.sync_copy(
        accum.at[pl.ds(s * ACC_PER_TILE, ACC_PER_TILE)],
        out_hbm.at[c, pl.ds(s * ACC_PER_TILE, ACC_PER_TILE)],
    )


_sc_kernel = functools.partial(
    pl.kernel,
    out_type=jax.ShapeDtypeStruct((NC, ACC), jnp.float32),
    mesh=plsc.VectorSubcoreMesh(
        core_axis_name="c", subcore_axis_name="s", num_cores=NC, num_subcores=NS
    ),
    scratch_types=[
        pltpu.VMEM_SHARED((ACC,), jnp.float32),
        pltpu.VMEM((ACC_PER_TILE,), jnp.float32),
        pltpu.VMEM((ROWS_PER_TILE, ROW), jnp.int32),
        pltpu.VMEM((ROWS_PER_TILE, ROW), jnp.float32),
        pltpu.SemaphoreType.DMA,
    ],
)(_sc_body)


def _tc_add(x_ref, o_ref):
    o_ref[...] = x_ref[0] + x_ref[1]


@jax.jit
def kernel(energy, batch):
    pad = N_PAD - N_ATOMS
    e = jnp.concatenate([energy, jnp.zeros((pad,), jnp.float32)])
    b = jnp.concatenate([batch, jnp.arange(pad, dtype=jnp.int32) % N_SEG])
    partials = _sc_kernel(
        e.reshape(NW * ROWS_PER_TILE, ROW), b.reshape(NW * ROWS_PER_TILE, ROW)
    )
    out = pl.pallas_call(
        _tc_add,
        out_shape=jax.ShapeDtypeStruct((ACC // ROW, ROW), jnp.float32),
    )(partials.reshape(NC, ACC // ROW, ROW))
    return out.reshape(ACC)[:N_SEG]
